# baseline (speedup 1.0000x reference)
import functools

import jax
import jax.numpy as jnp
from jax import lax
from jax.experimental import pallas as pl
from jax.experimental.pallas import tpu as pltpu
from jax.experimental.pallas import tpu_sc as plsc

_NUM_CORES = 2
_NUM_SUBCORES = 16
_NW = _NUM_CORES * _NUM_SUBCORES
_M = 2
_NB = 8
_L = 2


@functools.lru_cache(maxsize=None)
def _build_gather(V, D, N, K):
  n_per_w = N // _NW
  n_steps = n_per_w // _M
  assert N % _NW == 0 and n_per_w % _M == 0
  assert n_steps % _NB == 0 and n_steps >= _NB + _L
  mesh = plsc.VectorSubcoreMesh(core_axis_name="c", subcore_axis_name="s")

  @functools.partial(
      pl.kernel,
      out_type=jax.ShapeDtypeStruct((N, K, D), jnp.float32),
      mesh=mesh,
      scratch_types=[
          pltpu.VMEM((n_per_w, K), jnp.int32),
          [pltpu.VMEM((_M, K, D), jnp.float32)] * _NB,
          [pltpu.SemaphoreType.DMA] * _NB,
          [pltpu.SemaphoreType.DMA] * _NB,
      ],
  )
  def gather_kernel(table_hbm, idx_hbm, out_hbm, idx_v, slabs, gsems, osems):
    wid = lax.axis_index("s") * _NUM_CORES + lax.axis_index("c")
    obase = wid * n_per_w

    pltpu.sync_copy(idx_hbm.at[pl.ds(obase, n_per_w)], idx_v)

    def fire_gathers(h, p):
      for t in range(_M):
        pltpu.async_copy(
            table_hbm.at[idx_v.at[h * _M + t]], slabs[p].at[t], gsems[p])

    def drain_gathers(p):
      pltpu.make_async_copy(
          out_hbm.at[pl.ds(0, _M)], slabs[p], gsems[p]).wait()

    def out_write(h, p):
      return pltpu.make_async_copy(
          slabs[p], out_hbm.at[pl.ds(obase + h * _M, _M)], osems[p])

    def drain_writes(p):
      pltpu.make_async_copy(
          out_hbm.at[pl.ds(0, _M)], slabs[p], osems[p]).wait()

    for h in range(_L):
      fire_gathers(h, h)

    for h in range(_NB - _L):
      fire_gathers(h + _L, h + _L)
      drain_gathers(h)
      out_write(h, h).start()

    @pl.loop(0, (n_steps - _NB) // _NB)
    def _(ho):
      for hh in range(_NB):
        h = (_NB - _L) + _NB * ho + hh
        p = (_NB - _L + hh) % _NB
        pn = hh
        drain_writes(pn)
        fire_gathers(h + _L, pn)
        drain_gathers(p)
        out_write(h, p).start()

    for h in range(n_steps - _L, n_steps):
      p = h % _NB
      drain_gathers(p)
      out_write(h, p).start()
    for h in range(n_steps - _NB, n_steps):
      drain_writes(h % _NB)

  return gather_kernel


def kernel(inputs, indices, axis):
  del axis
  V, D = inputs.shape
  N, K = indices.shape
  return _build_gather(V, D, N, K)(inputs, indices.astype(jnp.int32))
